# baseline (speedup 1.0000x reference)
import functools

import jax
import jax.numpy as jnp
from jax import lax
from jax.experimental import pallas as pl
from jax.experimental.pallas import tpu as pltpu
from jax.experimental.pallas import tpu_sc as plsc

N = 10000
E = 160000
D_IN = 256
D_H = 512

NC = 2
NS = 16
W_EDGE = 128
WPT = 80
EPAD = NS * WPT * W_EDGE
NPAD = 10240
ROWS_PT = NPAD // NS
NDEG = NPAD
DEG_PT = NDEG // NS
CHUNK = 128
NCHUNK = D_H // CHUNK
RBLK = 1024
GRID = NPAD // RBLK

_mesh = plsc.VectorSubcoreMesh(core_axis_name="c", subcore_axis_name="s")


@functools.partial(
    pl.kernel,
    mesh=_mesh,
    out_type=jax.ShapeDtypeStruct((NDEG,), jnp.float32),
    scratch_types=[
        pltpu.VMEM((WPT, W_EDGE), jnp.int32),
        pltpu.VMEM((W_EDGE,), jnp.float32),
        pltpu.VMEM((DEG_PT,), jnp.float32),
        pltpu.VMEM_SHARED((NDEG,), jnp.float32),
    ],
)
def _deg_kernel(dstw_hbm, deg_hbm, dst_v, ones_v, zer_v, acc_s):
    cid = lax.axis_index("c")
    sid = lax.axis_index("s")

    @pl.when(cid == 0)
    def _():
        for i in range(W_EDGE // 16):
            ones_v[pl.ds(i * 16, 16)] = jnp.full((16,), 1.0, jnp.float32)
        for i in range(DEG_PT // 16):
            zer_v[pl.ds(i * 16, 16)] = jnp.zeros((16,), jnp.float32)
        pltpu.sync_copy(zer_v, acc_s.at[pl.ds(sid * DEG_PT, DEG_PT)])
        pltpu.sync_copy(dstw_hbm.at[pl.ds(sid * WPT, WPT)], dst_v)
        plsc.subcore_barrier()

        def body(w, carry):
            pltpu.sync_copy(ones_v, acc_s.at[dst_v.at[w]], add=True)
            return carry

        lax.fori_loop(0, WPT, body, 0)
        plsc.subcore_barrier()
        pltpu.sync_copy(acc_s.at[pl.ds(sid * DEG_PT, DEG_PT)],
                        deg_hbm.at[pl.ds(sid * DEG_PT, DEG_PT)])


@functools.partial(
    pl.kernel,
    mesh=_mesh,
    out_type=[jax.ShapeDtypeStruct((NPAD, CHUNK), jnp.float32)] * NCHUNK,
    scratch_types=[
        pltpu.VMEM((16, W_EDGE), jnp.int32),
        pltpu.VMEM((16, W_EDGE), jnp.int32),
        pltpu.VMEM((W_EDGE, CHUNK), jnp.float32),
        pltpu.VMEM((W_EDGE, CHUNK), jnp.float32),
        pltpu.VMEM_SHARED((NPAD, CHUNK), jnp.float32),
        pltpu.SemaphoreType.DMA,
        pltpu.SemaphoreType.DMA,
    ],
)
def _prop_kernel(g0, g1, g2, g3, srcw_hbm, dstw_hbm,
                 a0, a1, a2, a3, src_v, dst_v, buf_a, buf_b, acc_s,
                 sem_a, sem_b):
    cid = lax.axis_index("c")
    sid = lax.axis_index("s")
    WB = 16
    NB = WPT // WB
    gs = (g0, g1, g2, g3)
    outs = (a0, a1, a2, a3)
    for c in range(NCHUNK):
        @pl.when(cid == c // 2)
        def _(c=c):
            g = gs[c]
            out = outs[c]
            pltpu.sync_copy(g.at[pl.ds(sid * ROWS_PT, ROWS_PT)],
                            acc_s.at[pl.ds(sid * ROWS_PT, ROWS_PT)])
            plsc.subcore_barrier()

            def outer(b, carry):
                base = sid * WPT + b * WB
                pltpu.sync_copy(srcw_hbm.at[pl.ds(base, WB)], src_v)
                pltpu.sync_copy(dstw_hbm.at[pl.ds(base, WB)], dst_v)
                pltpu.async_copy(g.at[src_v.at[0]], buf_a, sem_a)

                def body(i, carry2):
                    w = 2 * i
                    cp_b = pltpu.async_copy(
                        g.at[src_v.at[w + 1]], buf_b, sem_b)
                    pltpu.make_async_copy(
                        g.at[src_v.at[w]], buf_a, sem_a).wait()
                    pltpu.sync_copy(buf_a, acc_s.at[dst_v.at[w]], add=True)

                    @pl.when(w + 2 < WB)
                    def _():
                        pltpu.async_copy(g.at[src_v.at[w + 2]], buf_a, sem_a)

                    cp_b.wait()
                    pltpu.sync_copy(buf_b, acc_s.at[dst_v.at[w + 1]],
                                    add=True)
                    return carry2

                lax.fori_loop(0, WB // 2, body, 0)
                return carry

            lax.fori_loop(0, NB, outer, 0)
            plsc.subcore_barrier()
            pltpu.sync_copy(acc_s.at[pl.ds(sid * ROWS_PT, ROWS_PT)],
                            out.at[pl.ds(sid * ROWS_PT, ROWS_PT)])


def _dis(deg_blk):
    return lax.rsqrt(deg_blk + 1.0)


def _tc_first_body(deg_ref, x_ref, w_ref, *g_refs):
    dis = _dis(deg_ref[...])
    h = jnp.dot(x_ref[...], w_ref[...], preferred_element_type=jnp.float32)
    g = h * dis
    for c in range(NCHUNK):
        g_refs[c][...] = g[:, c * CHUNK:(c + 1) * CHUNK]


_tc_first = pl.pallas_call(
    _tc_first_body,
    grid=(GRID,),
    in_specs=[
        pl.BlockSpec((RBLK, 1), lambda i: (i, 0)),
        pl.BlockSpec((RBLK, D_IN), lambda i: (i, 0)),
        pl.BlockSpec((D_IN, D_H), lambda i: (0, 0)),
    ],
    out_specs=[pl.BlockSpec((RBLK, CHUNK), lambda i: (i, 0))] * NCHUNK,
    out_shape=[jax.ShapeDtypeStruct((NPAD, CHUNK), jnp.float32)] * NCHUNK,
)


def _make_tc_mid(with_residual):
    def body(*refs):
        if with_residual:
            (deg_ref, b_ref, a0, a1, a2, a3, r0, r1, r2, r3, w_ref,
             go0, go1, go2, go3, h0, h1, h2, h3) = refs
            rs = (r0, r1, r2, r3)
        else:
            (deg_ref, b_ref, a0, a1, a2, a3, w_ref,
             go0, go1, go2, go3, h0, h1, h2, h3) = refs
            rs = None
        avs = (a0, a1, a2, a3)
        gos = (go0, go1, go2, go3)
        hs = (h0, h1, h2, h3)
        dis = _dis(deg_ref[...])
        acc = jnp.zeros((RBLK, D_H), jnp.float32)
        for c in range(NCHUNK):
            v = jnp.maximum(
                avs[c][...] * dis + b_ref[:, c * CHUNK:(c + 1) * CHUNK], 0.0)
            if with_residual:
                v = v + rs[c][...]
            hs[c][...] = v
            acc = acc + jnp.dot(v, w_ref[c * CHUNK:(c + 1) * CHUNK, :],
                                preferred_element_type=jnp.float32)
        g = acc * dis
        for c in range(NCHUNK):
            gos[c][...] = g[:, c * CHUNK:(c + 1) * CHUNK]

    n_in = 4 + (NCHUNK if with_residual else 0)
    in_specs = (
        [pl.BlockSpec((RBLK, 1), lambda i: (i, 0)),
         pl.BlockSpec((1, D_H), lambda i: (0, 0))]
        + [pl.BlockSpec((RBLK, CHUNK), lambda i: (i, 0))] * NCHUNK
        + ([pl.BlockSpec((RBLK, CHUNK), lambda i: (i, 0))] * NCHUNK
           if with_residual else [])
        + [pl.BlockSpec((D_H, D_H), lambda i: (0, 0))]
    )
    return pl.pallas_call(
        body,
        grid=(GRID,),
        in_specs=in_specs,
        out_specs=[pl.BlockSpec((RBLK, CHUNK), lambda i: (i, 0))] * (2 * NCHUNK),
        out_shape=[jax.ShapeDtypeStruct((NPAD, CHUNK), jnp.float32)] * (2 * NCHUNK),
    )


_tc_mid_plain = _make_tc_mid(False)
_tc_mid_res = _make_tc_mid(True)


def _tc_final_body(deg_ref, b_ref, a0, a1, a2, a3, h0, h1, h2, h3, out_ref):
    dis = _dis(deg_ref[...])
    avs = (a0, a1, a2, a3)
    hs = (h0, h1, h2, h3)
    for c in range(NCHUNK):
        out_ref[:, c * CHUNK:(c + 1) * CHUNK] = (
            avs[c][...] * dis + b_ref[:, c * CHUNK:(c + 1) * CHUNK]
            + hs[c][...])


_tc_final = pl.pallas_call(
    _tc_final_body,
    grid=(GRID,),
    in_specs=(
        [pl.BlockSpec((RBLK, 1), lambda i: (i, 0)),
         pl.BlockSpec((1, D_H), lambda i: (0, 0))]
        + [pl.BlockSpec((RBLK, CHUNK), lambda i: (i, 0))] * (2 * NCHUNK)
    ),
    out_specs=pl.BlockSpec((RBLK, D_H), lambda i: (i, 0)),
    out_shape=jax.ShapeDtypeStruct((NPAD, D_H), jnp.float32),
)


def kernel(x, edge_index, W1, b1, W2, b2, W3, b3):
    src = edge_index[0].astype(jnp.int32)
    dst = edge_index[1].astype(jnp.int32)
    order = jnp.argsort(src)
    src = src[order]
    dst = dst[order]
    pad = EPAD - E
    src_p = jnp.concatenate([src, jnp.zeros((pad,), jnp.int32)])
    dst_p = jnp.concatenate(
        [dst, N + (jnp.arange(pad, dtype=jnp.int32) % NS)])
    srcw = src_p.reshape(NS * WPT, W_EDGE)
    dstw = dst_p.reshape(NS * WPT, W_EDGE)

    deg = _deg_kernel(dstw).reshape(NPAD, 1)
    xp = jnp.pad(x, ((0, NPAD - N), (0, 0)))

    b1r = b1.reshape(1, D_H)
    b2r = b2.reshape(1, D_H)
    b3r = b3.reshape(1, D_H)

    g1 = _tc_first(deg, xp, W1)
    a1 = _prop_kernel(*g1, srcw, dstw)
    g2_and_h1 = _tc_mid_plain(deg, b1r, *a1, W2)
    g2, h1 = g2_and_h1[:NCHUNK], g2_and_h1[NCHUNK:]
    a2 = _prop_kernel(*g2, srcw, dstw)
    g3_and_h2 = _tc_mid_res(deg, b2r, *a2, *h1, W3)
    g3, h2 = g3_and_h2[:NCHUNK], g3_and_h2[NCHUNK:]
    a3 = _prop_kernel(*g3, srcw, dstw)
    return _tc_final(deg, b3r, *a3, *h2)[:N]


# speedup vs baseline: 1.1257x; 1.1257x over previous
import functools

import jax
import jax.numpy as jnp
from jax import lax
from jax.experimental import pallas as pl
from jax.experimental.pallas import tpu as pltpu
from jax.experimental.pallas import tpu_sc as plsc

N = 10000
E = 160000
D_IN = 256
D_H = 512

NC = 2
NS = 16
W_EDGE = 64
WPT = 160
EPAD = NS * WPT * W_EDGE
NPAD = 10240
ROWS_PT = NPAD // NS
NDEG = NPAD
DEG_PT = NDEG // NS
CHUNK = 128
NCHUNK = D_H // CHUNK
RBLK = 1024
GRID = NPAD // RBLK

_mesh = plsc.VectorSubcoreMesh(core_axis_name="c", subcore_axis_name="s")


@functools.partial(
    pl.kernel,
    mesh=_mesh,
    out_type=jax.ShapeDtypeStruct((NDEG,), jnp.float32),
    scratch_types=[
        pltpu.VMEM((WPT, W_EDGE), jnp.int32),
        pltpu.VMEM((W_EDGE,), jnp.float32),
        pltpu.VMEM((DEG_PT,), jnp.float32),
        pltpu.VMEM_SHARED((NDEG,), jnp.float32),
    ],
)
def _deg_kernel(dstw_hbm, deg_hbm, dst_v, ones_v, zer_v, acc_s):
    cid = lax.axis_index("c")
    sid = lax.axis_index("s")

    @pl.when(cid == 0)
    def _():
        for i in range(W_EDGE // 16):
            ones_v[pl.ds(i * 16, 16)] = jnp.full((16,), 1.0, jnp.float32)
        for i in range(DEG_PT // 16):
            zer_v[pl.ds(i * 16, 16)] = jnp.zeros((16,), jnp.float32)
        pltpu.sync_copy(zer_v, acc_s.at[pl.ds(sid * DEG_PT, DEG_PT)])
        pltpu.sync_copy(dstw_hbm.at[pl.ds(sid * WPT, WPT)], dst_v)
        plsc.subcore_barrier()

        def body(w, carry):
            pltpu.sync_copy(ones_v, acc_s.at[dst_v.at[w]], add=True)
            return carry

        lax.fori_loop(0, WPT, body, 0)
        plsc.subcore_barrier()
        pltpu.sync_copy(acc_s.at[pl.ds(sid * DEG_PT, DEG_PT)],
                        deg_hbm.at[pl.ds(sid * DEG_PT, DEG_PT)])


@functools.partial(
    pl.kernel,
    mesh=_mesh,
    out_type=[jax.ShapeDtypeStruct((NPAD, CHUNK), jnp.float32)] * NCHUNK,
    scratch_types=[
        pltpu.VMEM((16, W_EDGE), jnp.int32),
        pltpu.VMEM((16, W_EDGE), jnp.int32),
        pltpu.VMEM((W_EDGE, CHUNK), jnp.float32),
        pltpu.VMEM((W_EDGE, CHUNK), jnp.float32),
        pltpu.VMEM((W_EDGE, CHUNK), jnp.float32),
        pltpu.VMEM((W_EDGE, CHUNK), jnp.float32),
        pltpu.VMEM_SHARED((NPAD, CHUNK), jnp.float32),
        pltpu.SemaphoreType.DMA,
        pltpu.SemaphoreType.DMA,
        pltpu.SemaphoreType.DMA,
        pltpu.SemaphoreType.DMA,
        pltpu.SemaphoreType.DMA,
        pltpu.SemaphoreType.DMA,
        pltpu.SemaphoreType.DMA,
        pltpu.SemaphoreType.DMA,
    ],
)
def _prop_kernel(g0, g1, g2, g3, srcw_hbm, dstw_hbm,
                 a0, a1, a2, a3, src_v, dst_v, b0, b1, b2, b3, acc_s,
                 sg0, sg1, sg2, sg3, ss0, ss1, ss2, ss3):
    cid = lax.axis_index("c")
    sid = lax.axis_index("s")
    WB = 16
    NB = WPT // WB
    bufs = (b0, b1, b2, b3)
    sg = (sg0, sg1, sg2, sg3)
    ss = (ss0, ss1, ss2, ss3)
    gs = (g0, g1, g2, g3)
    outs = (a0, a1, a2, a3)
    for c in range(NCHUNK):
        @pl.when(cid == c // 2)
        def _(c=c):
            g = gs[c]
            out = outs[c]
            pltpu.sync_copy(g.at[pl.ds(sid * ROWS_PT, ROWS_PT)],
                            acc_s.at[pl.ds(sid * ROWS_PT, ROWS_PT)])
            plsc.subcore_barrier()

            def outer(blk, carry):
                base = sid * WPT + blk * WB
                pltpu.sync_copy(srcw_hbm.at[pl.ds(base, WB)], src_v)
                pltpu.sync_copy(dstw_hbm.at[pl.ds(base, WB)], dst_v)
                pltpu.async_copy(g.at[src_v.at[0]], bufs[0], sg[0])
                pltpu.async_copy(g.at[src_v.at[1]], bufs[1], sg[1])

                def rounds(r, carry2):
                    for b in range(4):
                        w = 4 * r + b
                        pltpu.make_async_copy(
                            g.at[src_v.at[w]], bufs[b], sg[b]).wait()
                        pltpu.async_copy(bufs[b], acc_s.at[dst_v.at[w]],
                                         ss[b], add=True)
                        b2 = (b + 2) % 4
                        if b < 2:
                            @pl.when(r >= 1)
                            def _(b2=b2, w=w):
                                pltpu.make_async_copy(
                                    bufs[b2], acc_s.at[dst_v.at[w]],
                                    ss[b2]).wait()
                            pltpu.async_copy(
                                g.at[src_v.at[w + 2]], bufs[b2], sg[b2])
                        else:
                            pltpu.make_async_copy(
                                bufs[b2], acc_s.at[dst_v.at[w]],
                                ss[b2]).wait()

                            @pl.when(r < WB // 4 - 1)
                            def _(b2=b2, w=w):
                                pltpu.async_copy(
                                    g.at[src_v.at[w + 2]], bufs[b2], sg[b2])
                    return carry2

                lax.fori_loop(0, WB // 4, rounds, 0)
                pltpu.make_async_copy(
                    bufs[2], acc_s.at[dst_v.at[WB - 2]], ss[2]).wait()
                pltpu.make_async_copy(
                    bufs[3], acc_s.at[dst_v.at[WB - 1]], ss[3]).wait()
                return carry

            lax.fori_loop(0, NB, outer, 0)
            plsc.subcore_barrier()
            pltpu.sync_copy(acc_s.at[pl.ds(sid * ROWS_PT, ROWS_PT)],
                            out.at[pl.ds(sid * ROWS_PT, ROWS_PT)])


def _dis(deg_blk):
    return lax.rsqrt(deg_blk + 1.0)


def _tc_first_body(deg_ref, x_ref, w_ref, *g_refs):
    dis = _dis(deg_ref[...])
    h = jnp.dot(x_ref[...], w_ref[...], preferred_element_type=jnp.float32)
    g = h * dis
    for c in range(NCHUNK):
        g_refs[c][...] = g[:, c * CHUNK:(c + 1) * CHUNK]


_tc_first = pl.pallas_call(
    _tc_first_body,
    grid=(GRID,),
    in_specs=[
        pl.BlockSpec((RBLK, 1), lambda i: (i, 0)),
        pl.BlockSpec((RBLK, D_IN), lambda i: (i, 0)),
        pl.BlockSpec((D_IN, D_H), lambda i: (0, 0)),
    ],
    out_specs=[pl.BlockSpec((RBLK, CHUNK), lambda i: (i, 0))] * NCHUNK,
    out_shape=[jax.ShapeDtypeStruct((NPAD, CHUNK), jnp.float32)] * NCHUNK,
)


def _make_tc_mid(with_residual):
    def body(*refs):
        if with_residual:
            (deg_ref, b_ref, a0, a1, a2, a3, r0, r1, r2, r3, w_ref,
             go0, go1, go2, go3, h0, h1, h2, h3) = refs
            rs = (r0, r1, r2, r3)
        else:
            (deg_ref, b_ref, a0, a1, a2, a3, w_ref,
             go0, go1, go2, go3, h0, h1, h2, h3) = refs
            rs = None
        avs = (a0, a1, a2, a3)
        gos = (go0, go1, go2, go3)
        hs = (h0, h1, h2, h3)
        dis = _dis(deg_ref[...])
        acc = jnp.zeros((RBLK, D_H), jnp.float32)
        for c in range(NCHUNK):
            v = jnp.maximum(
                avs[c][...] * dis + b_ref[:, c * CHUNK:(c + 1) * CHUNK], 0.0)
            if with_residual:
                v = v + rs[c][...]
            hs[c][...] = v
            acc = acc + jnp.dot(v, w_ref[c * CHUNK:(c + 1) * CHUNK, :],
                                preferred_element_type=jnp.float32)
        g = acc * dis
        for c in range(NCHUNK):
            gos[c][...] = g[:, c * CHUNK:(c + 1) * CHUNK]

    n_in = 4 + (NCHUNK if with_residual else 0)
    in_specs = (
        [pl.BlockSpec((RBLK, 1), lambda i: (i, 0)),
         pl.BlockSpec((1, D_H), lambda i: (0, 0))]
        + [pl.BlockSpec((RBLK, CHUNK), lambda i: (i, 0))] * NCHUNK
        + ([pl.BlockSpec((RBLK, CHUNK), lambda i: (i, 0))] * NCHUNK
           if with_residual else [])
        + [pl.BlockSpec((D_H, D_H), lambda i: (0, 0))]
    )
    return pl.pallas_call(
        body,
        grid=(GRID,),
        in_specs=in_specs,
        out_specs=[pl.BlockSpec((RBLK, CHUNK), lambda i: (i, 0))] * (2 * NCHUNK),
        out_shape=[jax.ShapeDtypeStruct((NPAD, CHUNK), jnp.float32)] * (2 * NCHUNK),
    )


_tc_mid_plain = _make_tc_mid(False)
_tc_mid_res = _make_tc_mid(True)


def _tc_final_body(deg_ref, b_ref, a0, a1, a2, a3, h0, h1, h2, h3, out_ref):
    dis = _dis(deg_ref[...])
    avs = (a0, a1, a2, a3)
    hs = (h0, h1, h2, h3)
    for c in range(NCHUNK):
        out_ref[:, c * CHUNK:(c + 1) * CHUNK] = (
            avs[c][...] * dis + b_ref[:, c * CHUNK:(c + 1) * CHUNK]
            + hs[c][...])


_tc_final = pl.pallas_call(
    _tc_final_body,
    grid=(GRID,),
    in_specs=(
        [pl.BlockSpec((RBLK, 1), lambda i: (i, 0)),
         pl.BlockSpec((1, D_H), lambda i: (0, 0))]
        + [pl.BlockSpec((RBLK, CHUNK), lambda i: (i, 0))] * (2 * NCHUNK)
    ),
    out_specs=pl.BlockSpec((RBLK, D_H), lambda i: (i, 0)),
    out_shape=jax.ShapeDtypeStruct((NPAD, D_H), jnp.float32),
)


def kernel(x, edge_index, W1, b1, W2, b2, W3, b3):
    src = edge_index[0].astype(jnp.int32)
    dst = edge_index[1].astype(jnp.int32)
    pad = EPAD - E
    src_p = jnp.concatenate([src, jnp.zeros((pad,), jnp.int32)])
    dst_p = jnp.concatenate(
        [dst, N + (jnp.arange(pad, dtype=jnp.int32) % NS)])
    srcw = src_p.reshape(NS * WPT, W_EDGE)
    dstw = dst_p.reshape(NS * WPT, W_EDGE)

    deg = _deg_kernel(dstw).reshape(NPAD, 1)
    xp = jnp.pad(x, ((0, NPAD - N), (0, 0)))

    b1r = b1.reshape(1, D_H)
    b2r = b2.reshape(1, D_H)
    b3r = b3.reshape(1, D_H)

    g1 = _tc_first(deg, xp, W1)
    a1 = _prop_kernel(*g1, srcw, dstw)
    g2_and_h1 = _tc_mid_plain(deg, b1r, *a1, W2)
    g2, h1 = g2_and_h1[:NCHUNK], g2_and_h1[NCHUNK:]
    a2 = _prop_kernel(*g2, srcw, dstw)
    g3_and_h2 = _tc_mid_res(deg, b2r, *a2, *h1, W3)
    g3, h2 = g3_and_h2[:NCHUNK], g3_and_h2[NCHUNK:]
    a3 = _prop_kernel(*g3, srcw, dstw)
    return _tc_final(deg, b3r, *a3, *h2)[:N]


# speedup vs baseline: 1.1340x; 1.0074x over previous
import functools

import jax
import jax.numpy as jnp
from jax import lax
from jax.experimental import pallas as pl
from jax.experimental.pallas import tpu as pltpu
from jax.experimental.pallas import tpu_sc as plsc

N = 10000
E = 160000
D_IN = 256
D_H = 512

NC = 2
NS = 16
W_EDGE = 128
WPT = 80
EPAD = NS * WPT * W_EDGE
NPAD = 10240
ROWS_PT = NPAD // NS
NDEG = NPAD
DEG_PT = NDEG // NS
CHUNK = 128
NCHUNK = D_H // CHUNK
RBLK = 1024
GRID = NPAD // RBLK

_mesh = plsc.VectorSubcoreMesh(core_axis_name="c", subcore_axis_name="s")


@functools.partial(
    pl.kernel,
    mesh=_mesh,
    out_type=jax.ShapeDtypeStruct((NDEG,), jnp.float32),
    scratch_types=[
        pltpu.VMEM((WPT, W_EDGE), jnp.int32),
        pltpu.VMEM((W_EDGE,), jnp.float32),
        pltpu.VMEM((DEG_PT,), jnp.float32),
        pltpu.VMEM_SHARED((NDEG,), jnp.float32),
    ],
)
def _deg_kernel(dstw_hbm, deg_hbm, dst_v, ones_v, zer_v, acc_s):
    cid = lax.axis_index("c")
    sid = lax.axis_index("s")

    @pl.when(cid == 0)
    def _():
        for i in range(W_EDGE // 16):
            ones_v[pl.ds(i * 16, 16)] = jnp.full((16,), 1.0, jnp.float32)
        for i in range(DEG_PT // 16):
            zer_v[pl.ds(i * 16, 16)] = jnp.zeros((16,), jnp.float32)
        pltpu.sync_copy(zer_v, acc_s.at[pl.ds(sid * DEG_PT, DEG_PT)])
        pltpu.sync_copy(dstw_hbm.at[pl.ds(sid * WPT, WPT)], dst_v)
        plsc.subcore_barrier()

        def body(w, carry):
            pltpu.sync_copy(ones_v, acc_s.at[dst_v.at[w]], add=True)
            return carry

        lax.fori_loop(0, WPT, body, 0)
        plsc.subcore_barrier()
        pltpu.sync_copy(acc_s.at[pl.ds(sid * DEG_PT, DEG_PT)],
                        deg_hbm.at[pl.ds(sid * DEG_PT, DEG_PT)])


@functools.partial(
    pl.kernel,
    mesh=_mesh,
    out_type=[jax.ShapeDtypeStruct((NPAD, CHUNK), jnp.float32)] * NCHUNK,
    scratch_types=[
        pltpu.VMEM((16, W_EDGE), jnp.int32),
        pltpu.VMEM((16, W_EDGE), jnp.int32),
        pltpu.VMEM((W_EDGE, CHUNK), jnp.float32),
        pltpu.VMEM((W_EDGE, CHUNK), jnp.float32),
        pltpu.VMEM_SHARED((NPAD, CHUNK), jnp.float32),
        pltpu.SemaphoreType.DMA,
        pltpu.SemaphoreType.DMA,
    ],
)
def _prop_kernel(g0, g1, g2, g3, srcw_hbm, dstw_hbm,
                 a0, a1, a2, a3, src_v, dst_v, buf_a, buf_b, acc_s,
                 sem_a, sem_b):
    cid = lax.axis_index("c")
    sid = lax.axis_index("s")
    WB = 16
    NB = WPT // WB
    gs = (g0, g1, g2, g3)
    outs = (a0, a1, a2, a3)
    for c in range(NCHUNK):
        @pl.when(cid == c // 2)
        def _(c=c):
            g = gs[c]
            out = outs[c]
            pltpu.sync_copy(g.at[pl.ds(sid * ROWS_PT, ROWS_PT)],
                            acc_s.at[pl.ds(sid * ROWS_PT, ROWS_PT)])
            plsc.subcore_barrier()

            def outer(b, carry):
                base = sid * WPT + b * WB
                pltpu.sync_copy(srcw_hbm.at[pl.ds(base, WB)], src_v)
                pltpu.sync_copy(dstw_hbm.at[pl.ds(base, WB)], dst_v)
                pltpu.async_copy(g.at[src_v.at[0]], buf_a, sem_a)

                def body(i, carry2):
                    w = 2 * i
                    cp_b = pltpu.async_copy(
                        g.at[src_v.at[w + 1]], buf_b, sem_b)
                    pltpu.make_async_copy(
                        g.at[src_v.at[w]], buf_a, sem_a).wait()
                    pltpu.sync_copy(buf_a, acc_s.at[dst_v.at[w]], add=True)

                    @pl.when(w + 2 < WB)
                    def _():
                        pltpu.async_copy(g.at[src_v.at[w + 2]], buf_a, sem_a)

                    cp_b.wait()
                    pltpu.sync_copy(buf_b, acc_s.at[dst_v.at[w + 1]],
                                    add=True)
                    return carry2

                lax.fori_loop(0, WB // 2, body, 0)
                return carry

            lax.fori_loop(0, NB, outer, 0)
            plsc.subcore_barrier()
            pltpu.sync_copy(acc_s.at[pl.ds(sid * ROWS_PT, ROWS_PT)],
                            out.at[pl.ds(sid * ROWS_PT, ROWS_PT)])


def _dis(deg_blk):
    return lax.rsqrt(deg_blk + 1.0)


def _tc_first_body(deg_ref, x_ref, w_ref, *g_refs):
    dis = _dis(deg_ref[...])
    h = jnp.dot(x_ref[...], w_ref[...], preferred_element_type=jnp.float32)
    g = h * dis
    for c in range(NCHUNK):
        g_refs[c][...] = g[:, c * CHUNK:(c + 1) * CHUNK]


_tc_first = pl.pallas_call(
    _tc_first_body,
    grid=(GRID,),
    in_specs=[
        pl.BlockSpec((RBLK, 1), lambda i: (i, 0)),
        pl.BlockSpec((RBLK, D_IN), lambda i: (i, 0)),
        pl.BlockSpec((D_IN, D_H), lambda i: (0, 0)),
    ],
    out_specs=[pl.BlockSpec((RBLK, CHUNK), lambda i: (i, 0))] * NCHUNK,
    out_shape=[jax.ShapeDtypeStruct((NPAD, CHUNK), jnp.float32)] * NCHUNK,
)


def _make_tc_mid(with_residual):
    def body(*refs):
        if with_residual:
            (deg_ref, b_ref, a0, a1, a2, a3, r0, r1, r2, r3, w_ref,
             go0, go1, go2, go3, h0, h1, h2, h3) = refs
            rs = (r0, r1, r2, r3)
        else:
            (deg_ref, b_ref, a0, a1, a2, a3, w_ref,
             go0, go1, go2, go3, h0, h1, h2, h3) = refs
            rs = None
        avs = (a0, a1, a2, a3)
        gos = (go0, go1, go2, go3)
        hs = (h0, h1, h2, h3)
        dis = _dis(deg_ref[...])
        acc = jnp.zeros((RBLK, D_H), jnp.float32)
        for c in range(NCHUNK):
            v = jnp.maximum(
                avs[c][...] * dis + b_ref[:, c * CHUNK:(c + 1) * CHUNK], 0.0)
            if with_residual:
                v = v + rs[c][...]
            hs[c][...] = v
            acc = acc + jnp.dot(v, w_ref[c * CHUNK:(c + 1) * CHUNK, :],
                                preferred_element_type=jnp.float32)
        g = acc * dis
        for c in range(NCHUNK):
            gos[c][...] = g[:, c * CHUNK:(c + 1) * CHUNK]

    n_in = 4 + (NCHUNK if with_residual else 0)
    in_specs = (
        [pl.BlockSpec((RBLK, 1), lambda i: (i, 0)),
         pl.BlockSpec((1, D_H), lambda i: (0, 0))]
        + [pl.BlockSpec((RBLK, CHUNK), lambda i: (i, 0))] * NCHUNK
        + ([pl.BlockSpec((RBLK, CHUNK), lambda i: (i, 0))] * NCHUNK
           if with_residual else [])
        + [pl.BlockSpec((D_H, D_H), lambda i: (0, 0))]
    )
    return pl.pallas_call(
        body,
        grid=(GRID,),
        in_specs=in_specs,
        out_specs=[pl.BlockSpec((RBLK, CHUNK), lambda i: (i, 0))] * (2 * NCHUNK),
        out_shape=[jax.ShapeDtypeStruct((NPAD, CHUNK), jnp.float32)] * (2 * NCHUNK),
    )


_tc_mid_plain = _make_tc_mid(False)
_tc_mid_res = _make_tc_mid(True)


def _tc_final_body(deg_ref, b_ref, a0, a1, a2, a3, h0, h1, h2, h3, out_ref):
    dis = _dis(deg_ref[...])
    avs = (a0, a1, a2, a3)
    hs = (h0, h1, h2, h3)
    for c in range(NCHUNK):
        out_ref[:, c * CHUNK:(c + 1) * CHUNK] = (
            avs[c][...] * dis + b_ref[:, c * CHUNK:(c + 1) * CHUNK]
            + hs[c][...])


_tc_final = pl.pallas_call(
    _tc_final_body,
    grid=(GRID,),
    in_specs=(
        [pl.BlockSpec((RBLK, 1), lambda i: (i, 0)),
         pl.BlockSpec((1, D_H), lambda i: (0, 0))]
        + [pl.BlockSpec((RBLK, CHUNK), lambda i: (i, 0))] * (2 * NCHUNK)
    ),
    out_specs=pl.BlockSpec((RBLK, D_H), lambda i: (i, 0)),
    out_shape=jax.ShapeDtypeStruct((NPAD, D_H), jnp.float32),
)


def kernel(x, edge_index, W1, b1, W2, b2, W3, b3):
    src = edge_index[0].astype(jnp.int32)
    dst = edge_index[1].astype(jnp.int32)
    pad = EPAD - E
    src_p = jnp.concatenate([src, jnp.zeros((pad,), jnp.int32)])
    dst_p = jnp.concatenate(
        [dst, N + (jnp.arange(pad, dtype=jnp.int32) % NS)])
    srcw = src_p.reshape(NS * WPT, W_EDGE)
    dstw = dst_p.reshape(NS * WPT, W_EDGE)

    deg = _deg_kernel(dstw).reshape(NPAD, 1)
    xp = jnp.pad(x, ((0, NPAD - N), (0, 0)))

    b1r = b1.reshape(1, D_H)
    b2r = b2.reshape(1, D_H)
    b3r = b3.reshape(1, D_H)

    g1 = _tc_first(deg, xp, W1)
    a1 = _prop_kernel(*g1, srcw, dstw)
    g2_and_h1 = _tc_mid_plain(deg, b1r, *a1, W2)
    g2, h1 = g2_and_h1[:NCHUNK], g2_and_h1[NCHUNK:]
    a2 = _prop_kernel(*g2, srcw, dstw)
    g3_and_h2 = _tc_mid_res(deg, b2r, *a2, *h1, W3)
    g3, h2 = g3_and_h2[:NCHUNK], g3_and_h2[NCHUNK:]
    a3 = _prop_kernel(*g3, srcw, dstw)
    return _tc_final(deg, b3r, *a3, *h2)[:N]
